# Initial kernel scaffold; baseline (speedup 1.0000x reference)
import jax
import jax.numpy as jnp
from jax.experimental import pallas as pl


def kernel(x, edge_index, batch, W1, b1, W2, b2):
    raise NotImplementedError("write your pallas kernel here")


# speedup vs baseline: 21.3442x; 21.3442x over previous
import dataclasses
import functools

import jax
import jax.numpy as jnp
from jax import lax
from jax.experimental import pallas as pl
from jax.experimental.pallas import tpu as pltpu
from jax.experimental.pallas import tpu_sc as plsc

N = 10000
E = 320000
D_IN = 128
HID = 128
D_OUT = 64
G = 128

NC = 2
NS = 16
NW = NC * NS
EPW = E // NW
CH = 80
NCHUNK = EPW // CH
NPAD = 10240
RPT = NPAD // NS

_mesh = plsc.VectorSubcoreMesh(core_axis_name="c", subcore_axis_name="s")

_sc_params = pltpu.CompilerParams()
if "needs_layout_passes" in pltpu.CompilerParams.__dataclass_fields__:
    _sc_params = dataclasses.replace(_sc_params, needs_layout_passes=False)
_sc_flat_params = _sc_params
if "use_tc_tiling_on_sc" in pltpu.CompilerParams.__dataclass_fields__:
    _sc_flat_params = dataclasses.replace(_sc_flat_params,
                                          use_tc_tiling_on_sc=False)


def _deg_body(dst_hbm, out_hbm, idx_v, deg_v):
    cid = lax.axis_index("c")
    sid = lax.axis_index("s")
    wid = cid * NS + sid
    pltpu.sync_copy(dst_hbm.at[pl.ds(wid * EPW, EPW)], idx_v)
    zeros16 = jnp.zeros((16,), jnp.float32)

    @pl.loop(0, N // 16)
    def _zero(j):
        deg_v[pl.ds(j * 16, 16)] = zeros16

    ones16 = jnp.ones((16,), jnp.float32)

    @pl.loop(0, EPW // 16)
    def _count(j):
        idx16 = idx_v[pl.ds(j * 16, 16)]
        plsc.addupdate_scatter(deg_v, [idx16], ones16)

    pltpu.sync_copy(deg_v, out_hbm.at[pl.ds(wid * N, N)])


_deg_kernel = pl.kernel(
    out_type=jax.ShapeDtypeStruct((NW * N,), jnp.float32),
    mesh=_mesh,
    compiler_params=_sc_params,
    scratch_types=[
        pltpu.VMEM((EPW,), jnp.int32),
        pltpu.VMEM((N,), jnp.float32),
    ],
)(_deg_body)


def _edge_body(d, s_hbm, src_hbm, dst_hbm, zeros_hbm, out_hbm,
               src_v, dst_v, rows_v, acc_sh):
    cid = lax.axis_index("c")
    sid = lax.axis_index("s")
    wid = cid * NS + sid
    pltpu.sync_copy(zeros_hbm.at[pl.ds(sid * RPT, RPT)],
                    acc_sh.at[pl.ds(sid * RPT, RPT)])
    pltpu.sync_copy(src_hbm.at[wid], src_v)
    pltpu.sync_copy(dst_hbm.at[wid], dst_v)
    plsc.subcore_barrier()

    @pl.loop(0, NCHUNK)
    def _edges(j):
        pltpu.sync_copy(s_hbm.at[src_v.at[j]], rows_v)
        pltpu.sync_copy(rows_v, acc_sh.at[dst_v.at[j]], add=True)

    plsc.subcore_barrier()
    pltpu.sync_copy(acc_sh.at[pl.ds(sid * RPT, RPT)],
                    out_hbm.at[pl.ds(cid * NPAD + sid * RPT, RPT)])


def _make_edge_kernel(d):
    return pl.kernel(
        out_type=jax.ShapeDtypeStruct((NC * NPAD, d), jnp.float32),
        mesh=_mesh,
        compiler_params=_sc_flat_params,
        scratch_types=[
            pltpu.VMEM((NCHUNK, CH), jnp.int32),
            pltpu.VMEM((NCHUNK, CH), jnp.int32),
            pltpu.VMEM((CH, d), jnp.float32),
            pltpu.VMEM_SHARED((NPAD, d), jnp.float32),
        ],
    )(functools.partial(_edge_body, d))


_edge_kernel_h = _make_edge_kernel(HID)
_edge_kernel_o = _make_edge_kernel(D_OUT)


def _mm_body(x_ref, w_ref, o_ref):
    o_ref[...] = jnp.dot(x_ref[...], w_ref[...],
                         preferred_element_type=jnp.float32)


def _matmul(x, w, bm=1000):
    m, k = x.shape
    n = w.shape[1]
    return pl.pallas_call(
        _mm_body,
        grid=(m // bm,),
        in_specs=[pl.BlockSpec((bm, k), lambda i: (i, 0)),
                  pl.BlockSpec((k, n), lambda i: (0, 0))],
        out_specs=pl.BlockSpec((bm, n), lambda i: (i, 0)),
        out_shape=jax.ShapeDtypeStruct((m, n), jnp.float32),
    )(x, w)


def _dinv_body(parts_ref, dinv_ref):
    deg = jnp.sum(parts_ref[...], axis=0, keepdims=True) + 1.0
    dinv_ref[...] = lax.rsqrt(deg)


def _dinv_kernel(parts):
    return pl.pallas_call(
        _dinv_body,
        out_shape=jax.ShapeDtypeStruct((1, N), jnp.float32),
    )(parts)


def _scale_body(xw_ref, dinv_ref, s_ref):
    s_ref[...] = xw_ref[...] * dinv_ref[...]


def _scale_kernel(xw, dinv):
    return pl.pallas_call(
        _scale_body,
        out_shape=jax.ShapeDtypeStruct(xw.shape, jnp.float32),
    )(xw, dinv)


def _mid_body(acc_ref, s1_ref, dinv_ref, b1_ref, w2_ref, s2_ref):
    dinv = dinv_ref[...]
    acc = acc_ref[...]
    pre = (acc[0] + acc[1] + s1_ref[...]) * dinv + b1_ref[...]
    h1 = jnp.maximum(pre, 0.0)
    s2_ref[...] = jnp.dot(h1, w2_ref[...],
                          preferred_element_type=jnp.float32) * dinv


def _mid_kernel(acc1, s1, dinv, b1, w2):
    return pl.pallas_call(
        _mid_body,
        out_shape=jax.ShapeDtypeStruct((N, D_OUT), jnp.float32),
    )(acc1, s1, dinv, b1, w2)


def _out_body(acc_ref, s2_ref, dinv_ref, b2_ref, batch_ref, hs_ref, h_ref):
    acc = acc_ref[...]
    h = (acc[0] + acc[1] + s2_ref[...]) * dinv_ref[...] + b2_ref[...]
    h_ref[...] = h
    iota = lax.broadcasted_iota(jnp.int32, (G, N), 0)
    onehot_t = (batch_ref[...] == iota).astype(jnp.float32)
    hs_ref[...] = jnp.dot(onehot_t, h, preferred_element_type=jnp.float32)


def _out_kernel(acc2, s2, dinv, b2, batch_row):
    return pl.pallas_call(
        _out_body,
        out_shape=[jax.ShapeDtypeStruct((G, D_OUT), jnp.float32),
                   jax.ShapeDtypeStruct((N, D_OUT), jnp.float32)],
    )(acc2, s2, dinv, b2, batch_row)


def kernel(x, edge_index, batch, W1, b1, W2, b2):
    src3d = edge_index[0].reshape(NW, NCHUNK, CH)
    dst3d = edge_index[1].reshape(NW, NCHUNK, CH)
    dst_flat = edge_index[1]
    zeros_h = jnp.zeros((NPAD, HID), jnp.float32)
    zeros_o = jnp.zeros((NPAD, D_OUT), jnp.float32)

    xw1 = _matmul(x, W1)
    deg_parts = _deg_kernel(dst_flat).reshape(NW, N)
    dinv = _dinv_kernel(deg_parts).reshape(N, 1)
    s1 = _scale_kernel(xw1, dinv)
    acc1 = _edge_kernel_h(s1, src3d, dst3d, zeros_h)
    acc1 = acc1.reshape(NC, NPAD, HID)[:, :N, :]
    s2 = _mid_kernel(acc1, s1, dinv, b1.reshape(1, HID), W2)
    acc2 = _edge_kernel_o(s2, src3d, dst3d, zeros_o)
    acc2 = acc2.reshape(NC, NPAD, D_OUT)[:, :N, :]
    hs, h = _out_kernel(acc2, s2, dinv, b2.reshape(1, D_OUT),
                        batch.reshape(1, N))
    return (hs, h)


# speedup vs baseline: 27.2802x; 1.2781x over previous
import dataclasses
import functools

import jax
import jax.numpy as jnp
from jax import lax
from jax.experimental import pallas as pl
from jax.experimental.pallas import tpu as pltpu
from jax.experimental.pallas import tpu_sc as plsc

N = 10000
E = 320000
D_IN = 128
HID = 128
D_OUT = 64
G = 128

NC = 2
NS = 16
NW = NC * NS
EPW = E // NW
CH = 100
NCHUNK = EPW // CH
NPAD = 10240
RPT = NPAD // NS

_mesh = plsc.VectorSubcoreMesh(core_axis_name="c", subcore_axis_name="s")

_sc_params = pltpu.CompilerParams()
if "needs_layout_passes" in pltpu.CompilerParams.__dataclass_fields__:
    _sc_params = dataclasses.replace(_sc_params, needs_layout_passes=False)
_sc_flat_params = _sc_params
if "use_tc_tiling_on_sc" in pltpu.CompilerParams.__dataclass_fields__:
    _sc_flat_params = dataclasses.replace(_sc_flat_params,
                                          use_tc_tiling_on_sc=False)


def _deg_body(dst_hbm, out_hbm, idx_v, deg_v):
    cid = lax.axis_index("c")
    sid = lax.axis_index("s")
    wid = cid * NS + sid
    pltpu.sync_copy(dst_hbm.at[pl.ds(wid * EPW, EPW)], idx_v)
    zeros16 = jnp.zeros((16,), jnp.float32)

    @pl.loop(0, N // 16)
    def _zero(j):
        deg_v[pl.ds(j * 16, 16)] = zeros16

    ones16 = jnp.ones((16,), jnp.float32)

    @pl.loop(0, EPW // 16)
    def _count(j):
        idx16 = idx_v[pl.ds(j * 16, 16)]
        plsc.addupdate_scatter(deg_v, [idx16], ones16)

    pltpu.sync_copy(deg_v, out_hbm.at[pl.ds(wid * N, N)])


_deg_kernel = pl.kernel(
    out_type=jax.ShapeDtypeStruct((NW * N,), jnp.float32),
    mesh=_mesh,
    compiler_params=_sc_params,
    scratch_types=[
        pltpu.VMEM((EPW,), jnp.int32),
        pltpu.VMEM((N,), jnp.float32),
    ],
)(_deg_body)


def _edge_body(d, s_hbm, src_hbm, dst_hbm, zeros_hbm, out_hbm,
               src_v, dst_v, rows_a, rows_b, acc_sh, sem_a, sem_b):
    cid = lax.axis_index("c")
    sid = lax.axis_index("s")
    wid = cid * NS + sid
    pltpu.sync_copy(zeros_hbm.at[pl.ds(sid * RPT, RPT)],
                    acc_sh.at[pl.ds(sid * RPT, RPT)])
    pltpu.sync_copy(src_hbm.at[wid], src_v)
    pltpu.sync_copy(dst_hbm.at[wid], dst_v)
    plsc.subcore_barrier()

    pltpu.async_copy(s_hbm.at[src_v.at[0]], rows_a, sem_a)

    @pl.loop(0, NCHUNK // 2)
    def _edges(p):
        j = 2 * p
        pltpu.make_async_copy(s_hbm.at[src_v.at[j]], rows_a, sem_a).wait()
        pltpu.async_copy(s_hbm.at[src_v.at[j + 1]], rows_b, sem_b)
        pltpu.sync_copy(rows_a, acc_sh.at[dst_v.at[j]], add=True)
        pltpu.make_async_copy(s_hbm.at[src_v.at[j + 1]], rows_b, sem_b).wait()

        @pl.when(j + 2 < NCHUNK)
        def _next():
            pltpu.async_copy(s_hbm.at[src_v.at[j + 2]], rows_a, sem_a)

        pltpu.sync_copy(rows_b, acc_sh.at[dst_v.at[j + 1]], add=True)

    plsc.subcore_barrier()
    pltpu.sync_copy(acc_sh.at[pl.ds(sid * RPT, RPT)],
                    out_hbm.at[pl.ds(cid * NPAD + sid * RPT, RPT)])


def _make_edge_kernel(d):
    return pl.kernel(
        out_type=jax.ShapeDtypeStruct((NC * NPAD, d), jnp.float32),
        mesh=_mesh,
        compiler_params=_sc_flat_params,
        scratch_types=[
            pltpu.VMEM((NCHUNK, CH), jnp.int32),
            pltpu.VMEM((NCHUNK, CH), jnp.int32),
            pltpu.VMEM((CH, d), jnp.float32),
            pltpu.VMEM((CH, d), jnp.float32),
            pltpu.VMEM_SHARED((NPAD, d), jnp.float32),
            pltpu.SemaphoreType.DMA,
            pltpu.SemaphoreType.DMA,
        ],
    )(functools.partial(_edge_body, d))


_edge_kernel_h = _make_edge_kernel(HID)
_edge_kernel_o = _make_edge_kernel(D_OUT)


def _mm_body(x_ref, w_ref, o_ref):
    o_ref[...] = jnp.dot(x_ref[...], w_ref[...],
                         preferred_element_type=jnp.float32)


def _matmul(x, w, bm=1000):
    m, k = x.shape
    n = w.shape[1]
    return pl.pallas_call(
        _mm_body,
        grid=(m // bm,),
        in_specs=[pl.BlockSpec((bm, k), lambda i: (i, 0)),
                  pl.BlockSpec((k, n), lambda i: (0, 0))],
        out_specs=pl.BlockSpec((bm, n), lambda i: (i, 0)),
        out_shape=jax.ShapeDtypeStruct((m, n), jnp.float32),
    )(x, w)


def _dinv_body(parts_ref, dinv_ref):
    deg = jnp.sum(parts_ref[...], axis=0, keepdims=True) + 1.0
    dinv_ref[...] = lax.rsqrt(deg)


def _dinv_kernel(parts):
    return pl.pallas_call(
        _dinv_body,
        out_shape=jax.ShapeDtypeStruct((1, N), jnp.float32),
    )(parts)


def _scale_body(xw_ref, dinv_ref, s_ref):
    s_ref[...] = xw_ref[...] * dinv_ref[...]


def _scale_kernel(xw, dinv):
    return pl.pallas_call(
        _scale_body,
        out_shape=jax.ShapeDtypeStruct(xw.shape, jnp.float32),
    )(xw, dinv)


def _mid_body(acc_ref, s1_ref, dinv_ref, b1_ref, w2_ref, s2_ref):
    dinv = dinv_ref[...]
    acc = acc_ref[...]
    pre = (acc[0] + acc[1] + s1_ref[...]) * dinv + b1_ref[...]
    h1 = jnp.maximum(pre, 0.0)
    s2_ref[...] = jnp.dot(h1, w2_ref[...],
                          preferred_element_type=jnp.float32) * dinv


def _mid_kernel(acc1, s1, dinv, b1, w2):
    return pl.pallas_call(
        _mid_body,
        out_shape=jax.ShapeDtypeStruct((N, D_OUT), jnp.float32),
    )(acc1, s1, dinv, b1, w2)


def _out_body(acc_ref, s2_ref, dinv_ref, b2_ref, batch_ref, hs_ref, h_ref):
    acc = acc_ref[...]
    h = (acc[0] + acc[1] + s2_ref[...]) * dinv_ref[...] + b2_ref[...]
    h_ref[...] = h
    iota = lax.broadcasted_iota(jnp.int32, (G, N), 0)
    onehot_t = (batch_ref[...] == iota).astype(jnp.float32)
    hs_ref[...] = jnp.dot(onehot_t, h, preferred_element_type=jnp.float32)


def _out_kernel(acc2, s2, dinv, b2, batch_row):
    return pl.pallas_call(
        _out_body,
        out_shape=[jax.ShapeDtypeStruct((G, D_OUT), jnp.float32),
                   jax.ShapeDtypeStruct((N, D_OUT), jnp.float32)],
    )(acc2, s2, dinv, b2, batch_row)


def kernel(x, edge_index, batch, W1, b1, W2, b2):
    src3d = edge_index[0].reshape(NW, NCHUNK, CH)
    dst3d = edge_index[1].reshape(NW, NCHUNK, CH)
    dst_flat = edge_index[1]
    zeros_h = jnp.zeros((NPAD, HID), jnp.float32)
    zeros_o = jnp.zeros((NPAD, D_OUT), jnp.float32)

    xw1 = _matmul(x, W1)
    deg_parts = _deg_kernel(dst_flat).reshape(NW, N)
    dinv = _dinv_kernel(deg_parts).reshape(N, 1)
    s1 = _scale_kernel(xw1, dinv)
    acc1 = _edge_kernel_h(s1, src3d, dst3d, zeros_h)
    acc1 = acc1.reshape(NC, NPAD, HID)[:, :N, :]
    s2 = _mid_kernel(acc1, s1, dinv, b1.reshape(1, HID), W2)
    acc2 = _edge_kernel_o(s2, src3d, dst3d, zeros_o)
    acc2 = acc2.reshape(NC, NPAD, D_OUT)[:, :N, :]
    hs, h = _out_kernel(acc2, s2, dinv, b2.reshape(1, D_OUT),
                        batch.reshape(1, N))
    return (hs, h)


# speedup vs baseline: 30.1285x; 1.1044x over previous
import dataclasses
import functools

import jax
import jax.numpy as jnp
from jax import lax
from jax.experimental import pallas as pl
from jax.experimental.pallas import tpu as pltpu
from jax.experimental.pallas import tpu_sc as plsc

N = 10000
E = 320000
D_IN = 128
HID = 128
D_OUT = 64
G = 128

NC = 2
NS = 16
NW = NC * NS
EPW = E // NW
CH = 100
NCHUNK = EPW // CH
NPAD = 10240
RPT = NPAD // NS

_mesh = plsc.VectorSubcoreMesh(core_axis_name="c", subcore_axis_name="s")

_sc_params = pltpu.CompilerParams()
if "needs_layout_passes" in pltpu.CompilerParams.__dataclass_fields__:
    _sc_params = dataclasses.replace(_sc_params, needs_layout_passes=False)
_sc_flat_params = _sc_params
if "use_tc_tiling_on_sc" in pltpu.CompilerParams.__dataclass_fields__:
    _sc_flat_params = dataclasses.replace(_sc_flat_params,
                                          use_tc_tiling_on_sc=False)


def _deg_body(dst_hbm, out_hbm, idx_v, deg_v):
    cid = lax.axis_index("c")
    sid = lax.axis_index("s")
    wid = cid * NS + sid
    pltpu.sync_copy(dst_hbm.at[pl.ds(wid * EPW, EPW)], idx_v)
    zeros16 = jnp.zeros((16,), jnp.float32)

    @pl.loop(0, N // 16)
    def _zero(j):
        deg_v[pl.ds(j * 16, 16)] = zeros16

    ones16 = jnp.ones((16,), jnp.float32)

    @pl.loop(0, EPW // 16)
    def _count(j):
        idx16 = idx_v[pl.ds(j * 16, 16)]
        plsc.addupdate_scatter(deg_v, [idx16], ones16)

    pltpu.sync_copy(deg_v, out_hbm.at[pl.ds(wid * N, N)])


_deg_kernel = pl.kernel(
    out_type=jax.ShapeDtypeStruct((NW * N,), jnp.float32),
    mesh=_mesh,
    compiler_params=_sc_params,
    scratch_types=[
        pltpu.VMEM((EPW,), jnp.int32),
        pltpu.VMEM((N,), jnp.float32),
    ],
)(_deg_body)


def _edge_body(d, s_hbm, src_hbm, dst_hbm, out_hbm,
               src_v, dst_v, rows_a, rows_b, acc_sh, sem_a, sem_b):
    cid = lax.axis_index("c")
    sid = lax.axis_index("s")
    wid = cid * NS + sid
    zeros16 = jnp.zeros((16,), jnp.float32)

    @pl.loop(0, CH)
    def _zr(r):
        @pl.loop(0, d // 16)
        def _zc(c):
            rows_a[r, pl.ds(c * 16, 16)] = zeros16

    for k in range(RPT // CH):
        pltpu.sync_copy(rows_a, acc_sh.at[pl.ds(sid * RPT + k * CH, CH)])
    rem = RPT % CH
    pltpu.sync_copy(rows_a.at[pl.ds(0, rem)],
                    acc_sh.at[pl.ds(sid * RPT + RPT - rem, rem)])
    pltpu.sync_copy(src_hbm.at[wid], src_v)
    pltpu.sync_copy(dst_hbm.at[wid], dst_v)
    plsc.subcore_barrier()

    pltpu.async_copy(s_hbm.at[src_v.at[0]], rows_a, sem_a)

    @pl.loop(0, NCHUNK // 2)
    def _edges(p):
        j = 2 * p
        pltpu.make_async_copy(s_hbm.at[src_v.at[j]], rows_a, sem_a).wait()
        pltpu.async_copy(s_hbm.at[src_v.at[j + 1]], rows_b, sem_b)
        pltpu.sync_copy(rows_a, acc_sh.at[dst_v.at[j]], add=True)
        pltpu.make_async_copy(s_hbm.at[src_v.at[j + 1]], rows_b, sem_b).wait()

        @pl.when(j + 2 < NCHUNK)
        def _next():
            pltpu.async_copy(s_hbm.at[src_v.at[j + 2]], rows_a, sem_a)

        pltpu.sync_copy(rows_b, acc_sh.at[dst_v.at[j + 1]], add=True)

    plsc.subcore_barrier()
    pltpu.sync_copy(acc_sh.at[pl.ds(sid * RPT, RPT)],
                    out_hbm.at[pl.ds(cid * NPAD + sid * RPT, RPT)])


def _make_edge_kernel(d):
    return pl.kernel(
        out_type=jax.ShapeDtypeStruct((NC * NPAD, d), jnp.float32),
        mesh=_mesh,
        compiler_params=_sc_flat_params,
        scratch_types=[
            pltpu.VMEM((NCHUNK, CH), jnp.int32),
            pltpu.VMEM((NCHUNK, CH), jnp.int32),
            pltpu.VMEM((CH, d), jnp.float32),
            pltpu.VMEM((CH, d), jnp.float32),
            pltpu.VMEM_SHARED((NPAD, d), jnp.float32),
            pltpu.SemaphoreType.DMA,
            pltpu.SemaphoreType.DMA,
        ],
    )(functools.partial(_edge_body, d))


_edge_kernel_h = _make_edge_kernel(HID)
_edge_kernel_o = _make_edge_kernel(D_OUT)


def _prep_body(x_ref, w_ref, parts_ref, s_ref, dinv_ref):
    xw = jnp.dot(x_ref[...], w_ref[...], preferred_element_type=jnp.float32)
    ones_col = jnp.ones((NW, 1), jnp.float32)
    deg = lax.dot_general(parts_ref[...], ones_col, (((0,), (0,)), ((), ())),
                          preferred_element_type=jnp.float32) + 1.0
    dinv = lax.rsqrt(deg)
    dinv_ref[...] = dinv
    s_ref[...] = xw * dinv


def _prep_kernel(x, w1, parts):
    return pl.pallas_call(
        _prep_body,
        out_shape=[jax.ShapeDtypeStruct((N, HID), jnp.float32),
                   jax.ShapeDtypeStruct((N, 1), jnp.float32)],
    )(x, w1, parts)


def _mid_body(acc_ref, s1_ref, dinv_ref, b1_ref, w2_ref, s2_ref):
    dinv = dinv_ref[...]
    acc = acc_ref[...]
    pre = (acc[0:N] + acc[NPAD:NPAD + N] + s1_ref[...]) * dinv + b1_ref[...]
    h1 = jnp.maximum(pre, 0.0)
    s2_ref[...] = jnp.dot(h1, w2_ref[...],
                          preferred_element_type=jnp.float32) * dinv


def _mid_kernel(acc1, s1, dinv, b1, w2):
    return pl.pallas_call(
        _mid_body,
        out_shape=jax.ShapeDtypeStruct((N, D_OUT), jnp.float32),
    )(acc1, s1, dinv, b1, w2)


def _out_body(acc_ref, s2_ref, dinv_ref, b2_ref, batch_ref, hs_ref, h_ref):
    acc = acc_ref[...]
    h = (acc[0:N] + acc[NPAD:NPAD + N] + s2_ref[...]) * dinv_ref[...] \
        + b2_ref[...]
    h_ref[...] = h
    iota = lax.broadcasted_iota(jnp.int32, (G, N), 0)
    onehot_t = (batch_ref[...] == iota).astype(jnp.float32)
    hs_ref[...] = jnp.dot(onehot_t, h, preferred_element_type=jnp.float32)


def _out_kernel(acc2, s2, dinv, b2, batch_row):
    return pl.pallas_call(
        _out_body,
        out_shape=[jax.ShapeDtypeStruct((G, D_OUT), jnp.float32),
                   jax.ShapeDtypeStruct((N, D_OUT), jnp.float32)],
    )(acc2, s2, dinv, b2, batch_row)


def kernel(x, edge_index, batch, W1, b1, W2, b2):
    src3d = edge_index[0].reshape(NW, NCHUNK, CH)
    dst3d = edge_index[1].reshape(NW, NCHUNK, CH)
    dst_flat = edge_index[1]

    deg_parts = _deg_kernel(dst_flat).reshape(NW, N)
    s1, dinv = _prep_kernel(x, W1, deg_parts)
    acc1 = _edge_kernel_h(s1, src3d, dst3d)
    s2 = _mid_kernel(acc1, s1, dinv, b1.reshape(1, HID), W2)
    acc2 = _edge_kernel_o(s2, src3d, dst3d)
    hs, h = _out_kernel(acc2, s2, dinv, b2.reshape(1, D_OUT),
                        batch.reshape(1, N))
    return (hs, h)


# speedup vs baseline: 32.1132x; 1.0659x over previous
import dataclasses
import functools

import jax
import jax.numpy as jnp
from jax import lax
from jax.experimental import pallas as pl
from jax.experimental.pallas import tpu as pltpu
from jax.experimental.pallas import tpu_sc as plsc

N = 10000
E = 320000
D_IN = 128
HID = 128
D_OUT = 64
G = 128

NC = 2
NS = 16
NW = NC * NS
EPW = E // NW
CH = 100
NCHUNK = EPW // CH
NPAD = 10240
RPT = NPAD // NS

_mesh = plsc.VectorSubcoreMesh(core_axis_name="c", subcore_axis_name="s")

_sc_params = pltpu.CompilerParams()
if "needs_layout_passes" in pltpu.CompilerParams.__dataclass_fields__:
    _sc_params = dataclasses.replace(_sc_params, needs_layout_passes=False)
_sc_flat_params = _sc_params
if "use_tc_tiling_on_sc" in pltpu.CompilerParams.__dataclass_fields__:
    _sc_flat_params = dataclasses.replace(_sc_flat_params,
                                          use_tc_tiling_on_sc=False)


def _deg_body(dst_hbm, out_hbm, idx_v, deg_v):
    cid = lax.axis_index("c")
    sid = lax.axis_index("s")
    wid = cid * NS + sid
    pltpu.sync_copy(dst_hbm.at[pl.ds(wid * EPW, EPW)], idx_v)
    zeros16 = jnp.zeros((16,), jnp.float32)

    @pl.loop(0, N // 16)
    def _zero(j):
        deg_v[pl.ds(j * 16, 16)] = zeros16

    ones16 = jnp.ones((16,), jnp.float32)

    @pl.loop(0, EPW // 16)
    def _count(j):
        idx16 = idx_v[pl.ds(j * 16, 16)]
        plsc.addupdate_scatter(deg_v, [idx16], ones16)

    pltpu.sync_copy(deg_v, out_hbm.at[pl.ds(wid * N, N)])


_deg_kernel = pl.kernel(
    out_type=jax.ShapeDtypeStruct((NW * N,), jnp.float32),
    mesh=_mesh,
    compiler_params=_sc_params,
    scratch_types=[
        pltpu.VMEM((EPW,), jnp.int32),
        pltpu.VMEM((N,), jnp.float32),
    ],
)(_deg_body)


def _edge_body(d, s_hbm, src_hbm, dst_hbm, out_hbm,
               src_v, dst_v, rows_a, rows_b, acc_sh, sem_a, sem_b):
    cid = lax.axis_index("c")
    sid = lax.axis_index("s")
    wid = cid * NS + sid
    zeros16 = jnp.zeros((16,), jnp.float32)

    @pl.loop(0, CH)
    def _zr(r):
        @pl.loop(0, d // 16)
        def _zc(c):
            rows_a[r, pl.ds(c * 16, 16)] = zeros16

    for k in range(RPT // CH):
        pltpu.sync_copy(rows_a, acc_sh.at[pl.ds(sid * RPT + k * CH, CH)])
    rem = RPT % CH
    pltpu.sync_copy(rows_a.at[pl.ds(0, rem)],
                    acc_sh.at[pl.ds(sid * RPT + RPT - rem, rem)])
    pltpu.sync_copy(src_hbm.at[wid], src_v)
    pltpu.sync_copy(dst_hbm.at[wid], dst_v)
    plsc.subcore_barrier()

    pltpu.async_copy(s_hbm.at[src_v.at[0]], rows_a, sem_a)

    @pl.loop(0, NCHUNK // 2)
    def _edges(p):
        j = 2 * p
        pltpu.make_async_copy(s_hbm.at[src_v.at[j]], rows_a, sem_a).wait()
        pltpu.async_copy(s_hbm.at[src_v.at[j + 1]], rows_b, sem_b)
        pltpu.sync_copy(rows_a, acc_sh.at[dst_v.at[j]], add=True)
        pltpu.make_async_copy(s_hbm.at[src_v.at[j + 1]], rows_b, sem_b).wait()

        @pl.when(j + 2 < NCHUNK)
        def _next():
            pltpu.async_copy(s_hbm.at[src_v.at[j + 2]], rows_a, sem_a)

        pltpu.sync_copy(rows_b, acc_sh.at[dst_v.at[j + 1]], add=True)

    plsc.subcore_barrier()
    pltpu.sync_copy(acc_sh.at[pl.ds(sid * RPT, RPT)],
                    out_hbm.at[pl.ds(cid * NPAD + sid * RPT, RPT)])


def _make_edge_kernel(d):
    return pl.kernel(
        out_type=jax.ShapeDtypeStruct((NC * NPAD, d), jnp.float32),
        mesh=_mesh,
        compiler_params=_sc_flat_params,
        scratch_types=[
            pltpu.VMEM((NCHUNK, CH), jnp.int32),
            pltpu.VMEM((NCHUNK, CH), jnp.int32),
            pltpu.VMEM((CH, d), jnp.float32),
            pltpu.VMEM((CH, d), jnp.float32),
            pltpu.VMEM_SHARED((NPAD, d), jnp.float32),
            pltpu.SemaphoreType.DMA,
            pltpu.SemaphoreType.DMA,
        ],
    )(functools.partial(_edge_body, d))


def _edge_body_staged(d, s_hbm, src_hbm, dst_hbm, out_hbm,
                      src_v, dst_v, rows_a, rows_b, s_sh, acc_sh,
                      sem_a, sem_b):
    cid = lax.axis_index("c")
    sid = lax.axis_index("s")
    wid = cid * NS + sid
    zeros16 = jnp.zeros((16,), jnp.float32)

    @pl.loop(0, CH)
    def _zr(r):
        @pl.loop(0, d // 16)
        def _zc(c):
            rows_a[r, pl.ds(c * 16, 16)] = zeros16

    for k in range(RPT // CH):
        pltpu.sync_copy(rows_a, acc_sh.at[pl.ds(sid * RPT + k * CH, CH)])
    rem = RPT % CH
    pltpu.sync_copy(rows_a.at[pl.ds(0, rem)],
                    acc_sh.at[pl.ds(sid * RPT + RPT - rem, rem)])
    pltpu.sync_copy(s_hbm.at[pl.ds(sid * (N // NS), N // NS)],
                    s_sh.at[pl.ds(sid * (N // NS), N // NS)])
    pltpu.sync_copy(src_hbm.at[wid], src_v)
    pltpu.sync_copy(dst_hbm.at[wid], dst_v)
    plsc.subcore_barrier()

    pltpu.async_copy(s_sh.at[src_v.at[0]], rows_a, sem_a)

    @pl.loop(0, NCHUNK // 2)
    def _edges(p):
        j = 2 * p
        pltpu.make_async_copy(s_sh.at[src_v.at[j]], rows_a, sem_a).wait()
        pltpu.async_copy(s_sh.at[src_v.at[j + 1]], rows_b, sem_b)
        pltpu.sync_copy(rows_a, acc_sh.at[dst_v.at[j]], add=True)
        pltpu.make_async_copy(s_sh.at[src_v.at[j + 1]], rows_b, sem_b).wait()

        @pl.when(j + 2 < NCHUNK)
        def _next():
            pltpu.async_copy(s_sh.at[src_v.at[j + 2]], rows_a, sem_a)

        pltpu.sync_copy(rows_b, acc_sh.at[dst_v.at[j + 1]], add=True)

    plsc.subcore_barrier()
    pltpu.sync_copy(acc_sh.at[pl.ds(sid * RPT, RPT)],
                    out_hbm.at[pl.ds(cid * NPAD + sid * RPT, RPT)])


def _make_edge_kernel_staged(d):
    return pl.kernel(
        out_type=jax.ShapeDtypeStruct((NC * NPAD, d), jnp.float32),
        mesh=_mesh,
        compiler_params=_sc_flat_params,
        scratch_types=[
            pltpu.VMEM((NCHUNK, CH), jnp.int32),
            pltpu.VMEM((NCHUNK, CH), jnp.int32),
            pltpu.VMEM((CH, d), jnp.float32),
            pltpu.VMEM((CH, d), jnp.float32),
            pltpu.VMEM_SHARED((N, d), jnp.float32),
            pltpu.VMEM_SHARED((NPAD, d), jnp.float32),
            pltpu.SemaphoreType.DMA,
            pltpu.SemaphoreType.DMA,
        ],
    )(functools.partial(_edge_body_staged, d))


_edge_kernel_h = _make_edge_kernel(HID)
_edge_kernel_o = _make_edge_kernel_staged(D_OUT)


def _prep_body(x_ref, w_ref, parts_ref, s_ref, dinv_ref):
    xw = jnp.dot(x_ref[...], w_ref[...], preferred_element_type=jnp.float32)
    ones_col = jnp.ones((NW, 1), jnp.float32)
    deg = lax.dot_general(parts_ref[...], ones_col, (((0,), (0,)), ((), ())),
                          preferred_element_type=jnp.float32) + 1.0
    dinv = lax.rsqrt(deg)
    dinv_ref[...] = dinv
    s_ref[...] = xw * dinv


def _prep_kernel(x, w1, parts):
    return pl.pallas_call(
        _prep_body,
        out_shape=[jax.ShapeDtypeStruct((N, HID), jnp.float32),
                   jax.ShapeDtypeStruct((N, 1), jnp.float32)],
    )(x, w1, parts)


def _mid_body(acc_ref, s1_ref, dinv_ref, b1_ref, w2_ref, s2_ref):
    dinv = dinv_ref[...]
    acc = acc_ref[...]
    pre = (acc[0:N] + acc[NPAD:NPAD + N] + s1_ref[...]) * dinv + b1_ref[...]
    h1 = jnp.maximum(pre, 0.0)
    s2_ref[...] = jnp.dot(h1, w2_ref[...],
                          preferred_element_type=jnp.float32) * dinv


def _mid_kernel(acc1, s1, dinv, b1, w2):
    return pl.pallas_call(
        _mid_body,
        out_shape=jax.ShapeDtypeStruct((N, D_OUT), jnp.float32),
    )(acc1, s1, dinv, b1, w2)


def _out_body(acc_ref, s2_ref, dinv_ref, b2_ref, batch_ref, hs_ref, h_ref):
    acc = acc_ref[...]
    h = (acc[0:N] + acc[NPAD:NPAD + N] + s2_ref[...]) * dinv_ref[...] \
        + b2_ref[...]
    h_ref[...] = h
    iota = lax.broadcasted_iota(jnp.int32, (G, N), 0)
    onehot_t = (batch_ref[...] == iota).astype(jnp.float32)
    hs_ref[...] = jnp.dot(onehot_t, h, preferred_element_type=jnp.float32)


def _out_kernel(acc2, s2, dinv, b2, batch_row):
    return pl.pallas_call(
        _out_body,
        out_shape=[jax.ShapeDtypeStruct((G, D_OUT), jnp.float32),
                   jax.ShapeDtypeStruct((N, D_OUT), jnp.float32)],
    )(acc2, s2, dinv, b2, batch_row)


def kernel(x, edge_index, batch, W1, b1, W2, b2):
    src3d = edge_index[0].reshape(NW, NCHUNK, CH)
    dst3d = edge_index[1].reshape(NW, NCHUNK, CH)
    dst_flat = edge_index[1]

    deg_parts = _deg_kernel(dst_flat).reshape(NW, N)
    s1, dinv = _prep_kernel(x, W1, deg_parts)
    acc1 = _edge_kernel_h(s1, src3d, dst3d)
    s2 = _mid_kernel(acc1, s1, dinv, b1.reshape(1, HID), W2)
    acc2 = _edge_kernel_o(s2, src3d, dst3d)
    hs, h = _out_kernel(acc2, s2, dinv, b2.reshape(1, D_OUT),
                        batch.reshape(1, N))
    return (hs, h)


# speedup vs baseline: 35.1911x; 1.0958x over previous
import dataclasses
import functools

import jax
import jax.numpy as jnp
from jax import lax
from jax.experimental import pallas as pl
from jax.experimental.pallas import tpu as pltpu
from jax.experimental.pallas import tpu_sc as plsc

N = 10000
E = 320000
D_IN = 128
HID = 128
D_OUT = 64
G = 128

NC = 2
NS = 16
NW = NC * NS
EPW = E // NW
CH = 100
NCHUNK = EPW // CH
CH1 = 50
NCHUNK1 = EPW // CH1
NBUF = 4
NPAD = 10240
RPT = NPAD // NS

_mesh = plsc.VectorSubcoreMesh(core_axis_name="c", subcore_axis_name="s")

_sc_params = pltpu.CompilerParams()
if "needs_layout_passes" in pltpu.CompilerParams.__dataclass_fields__:
    _sc_params = dataclasses.replace(_sc_params, needs_layout_passes=False)
_sc_flat_params = _sc_params
if "use_tc_tiling_on_sc" in pltpu.CompilerParams.__dataclass_fields__:
    _sc_flat_params = dataclasses.replace(_sc_flat_params,
                                          use_tc_tiling_on_sc=False)


def _deg_body(dst_hbm, out_hbm, idx_v, deg_v):
    cid = lax.axis_index("c")
    sid = lax.axis_index("s")
    wid = cid * NS + sid
    pltpu.sync_copy(dst_hbm.at[pl.ds(wid * EPW, EPW)], idx_v)
    zeros16 = jnp.zeros((16,), jnp.float32)

    @pl.loop(0, N // 16)
    def _zero(j):
        deg_v[pl.ds(j * 16, 16)] = zeros16

    ones16 = jnp.ones((16,), jnp.float32)

    @pl.loop(0, EPW // 16)
    def _count(j):
        idx16 = idx_v[pl.ds(j * 16, 16)]
        plsc.addupdate_scatter(deg_v, [idx16], ones16)

    pltpu.sync_copy(deg_v, out_hbm.at[pl.ds(wid * N, N)])


_deg_kernel = pl.kernel(
    out_type=jax.ShapeDtypeStruct((NW * N,), jnp.float32),
    mesh=_mesh,
    compiler_params=_sc_params,
    scratch_types=[
        pltpu.VMEM((EPW,), jnp.int32),
        pltpu.VMEM((N,), jnp.float32),
    ],
)(_deg_body)


def _edge_body(d, ch, nchunk, s_hbm, src_hbm, dst_hbm, out_hbm,
               src_v, dst_v, rows, sems, acc_sh):
    cid = lax.axis_index("c")
    sid = lax.axis_index("s")
    wid = cid * NS + sid
    zeros16 = jnp.zeros((16,), jnp.float32)

    @pl.loop(0, ch)
    def _zr(r):
        @pl.loop(0, d // 16)
        def _zc(c):
            rows[0][r, pl.ds(c * 16, 16)] = zeros16

    for k in range(RPT // ch):
        pltpu.sync_copy(rows[0], acc_sh.at[pl.ds(sid * RPT + k * ch, ch)])
    rem = RPT % ch
    pltpu.sync_copy(rows[0].at[pl.ds(0, rem)],
                    acc_sh.at[pl.ds(sid * RPT + RPT - rem, rem)])
    pltpu.sync_copy(src_hbm.at[wid], src_v)
    pltpu.sync_copy(dst_hbm.at[wid], dst_v)
    plsc.subcore_barrier()

    nbuf = len(rows)
    for b in range(nbuf - 1):
        pltpu.async_copy(s_hbm.at[src_v.at[b]], rows[b], sems[b])

    @pl.loop(0, nchunk // nbuf)
    def _edges(p):
        j = p * nbuf
        for b in range(nbuf):
            pltpu.make_async_copy(s_hbm.at[src_v.at[j + b]],
                                  rows[b], sems[b]).wait()
            nxt = j + b + nbuf - 1
            bb = (b + nbuf - 1) % nbuf

            @pl.when(nxt < nchunk)
            def _issue():
                pltpu.async_copy(s_hbm.at[src_v.at[nxt]],
                                 rows[bb], sems[bb])

            pltpu.sync_copy(rows[b], acc_sh.at[dst_v.at[j + b]], add=True)

    for c in range((nchunk // nbuf) * nbuf, nchunk):
        pltpu.make_async_copy(s_hbm.at[src_v.at[c]],
                              rows[c % nbuf], sems[c % nbuf]).wait()
        pltpu.sync_copy(rows[c % nbuf], acc_sh.at[dst_v.at[c]], add=True)

    plsc.subcore_barrier()
    pltpu.sync_copy(acc_sh.at[pl.ds(sid * RPT, RPT)],
                    out_hbm.at[pl.ds(cid * NPAD + sid * RPT, RPT)])


def _make_edge_kernel(d):
    def body(s_hbm, src_hbm, dst_hbm, out_hbm, src_v, dst_v, *rest):
        rows = list(rest[:NBUF])
        acc_sh = rest[NBUF]
        sems = list(rest[NBUF + 1:NBUF + 1 + NBUF])
        _edge_body(d, CH1, NCHUNK1, s_hbm, src_hbm, dst_hbm, out_hbm,
                   src_v, dst_v, rows, sems, acc_sh)

    return pl.kernel(
        out_type=jax.ShapeDtypeStruct((NC * NPAD, d), jnp.float32),
        mesh=_mesh,
        compiler_params=_sc_flat_params,
        scratch_types=[
            pltpu.VMEM((NCHUNK1, CH1), jnp.int32),
            pltpu.VMEM((NCHUNK1, CH1), jnp.int32),
        ] + [pltpu.VMEM((CH1, d), jnp.float32) for _ in range(NBUF)] + [
            pltpu.VMEM_SHARED((NPAD, d), jnp.float32),
        ] + [pltpu.SemaphoreType.DMA for _ in range(NBUF)],
    )(body)


def _edge_body_staged(d, s_hbm, src_hbm, dst_hbm, out_hbm,
                      src_v, dst_v, rows_a, rows_b, s_sh, acc_sh,
                      sem_a, sem_b):
    cid = lax.axis_index("c")
    sid = lax.axis_index("s")
    wid = cid * NS + sid
    zeros16 = jnp.zeros((16,), jnp.float32)

    @pl.loop(0, CH)
    def _zr(r):
        @pl.loop(0, d // 16)
        def _zc(c):
            rows_a[r, pl.ds(c * 16, 16)] = zeros16

    for k in range(RPT // CH):
        pltpu.sync_copy(rows_a, acc_sh.at[pl.ds(sid * RPT + k * CH, CH)])
    rem = RPT % CH
    pltpu.sync_copy(rows_a.at[pl.ds(0, rem)],
                    acc_sh.at[pl.ds(sid * RPT + RPT - rem, rem)])
    pltpu.sync_copy(s_hbm.at[pl.ds(sid * (N // NS), N // NS)],
                    s_sh.at[pl.ds(sid * (N // NS), N // NS)])
    pltpu.sync_copy(src_hbm.at[wid], src_v)
    pltpu.sync_copy(dst_hbm.at[wid], dst_v)
    plsc.subcore_barrier()

    pltpu.async_copy(s_sh.at[src_v.at[0]], rows_a, sem_a)

    @pl.loop(0, NCHUNK // 2)
    def _edges(p):
        j = 2 * p
        pltpu.make_async_copy(s_sh.at[src_v.at[j]], rows_a, sem_a).wait()
        pltpu.async_copy(s_sh.at[src_v.at[j + 1]], rows_b, sem_b)
        pltpu.sync_copy(rows_a, acc_sh.at[dst_v.at[j]], add=True)
        pltpu.make_async_copy(s_sh.at[src_v.at[j + 1]], rows_b, sem_b).wait()

        @pl.when(j + 2 < NCHUNK)
        def _next():
            pltpu.async_copy(s_sh.at[src_v.at[j + 2]], rows_a, sem_a)

        pltpu.sync_copy(rows_b, acc_sh.at[dst_v.at[j + 1]], add=True)

    plsc.subcore_barrier()
    pltpu.sync_copy(acc_sh.at[pl.ds(sid * RPT, RPT)],
                    out_hbm.at[pl.ds(cid * NPAD + sid * RPT, RPT)])


def _make_edge_kernel_staged(d):
    return pl.kernel(
        out_type=jax.ShapeDtypeStruct((NC * NPAD, d), jnp.float32),
        mesh=_mesh,
        compiler_params=_sc_flat_params,
        scratch_types=[
            pltpu.VMEM((NCHUNK, CH), jnp.int32),
            pltpu.VMEM((NCHUNK, CH), jnp.int32),
            pltpu.VMEM((CH, d), jnp.float32),
            pltpu.VMEM((CH, d), jnp.float32),
            pltpu.VMEM_SHARED((N, d), jnp.float32),
            pltpu.VMEM_SHARED((NPAD, d), jnp.float32),
            pltpu.SemaphoreType.DMA,
            pltpu.SemaphoreType.DMA,
        ],
    )(functools.partial(_edge_body_staged, d))


_edge_kernel_h = _make_edge_kernel(HID)
_edge_kernel_o = _make_edge_kernel_staged(D_OUT)


def _prep_body(x_ref, w_ref, parts_ref, s_ref, dinv_ref):
    xw = jnp.dot(x_ref[...], w_ref[...], preferred_element_type=jnp.float32)
    ones_col = jnp.ones((NW, 1), jnp.float32)
    deg = lax.dot_general(parts_ref[...], ones_col, (((0,), (0,)), ((), ())),
                          preferred_element_type=jnp.float32) + 1.0
    dinv = lax.rsqrt(deg)
    dinv_ref[...] = dinv
    s_ref[...] = xw * dinv


def _prep_kernel(x, w1, parts):
    return pl.pallas_call(
        _prep_body,
        out_shape=[jax.ShapeDtypeStruct((N, HID), jnp.float32),
                   jax.ShapeDtypeStruct((N, 1), jnp.float32)],
    )(x, w1, parts)


def _mid_body(acc_ref, s1_ref, dinv_ref, b1_ref, w2_ref, s2_ref):
    dinv = dinv_ref[...]
    acc = acc_ref[...]
    pre = (acc[0:N] + acc[NPAD:NPAD + N] + s1_ref[...]) * dinv + b1_ref[...]
    h1 = jnp.maximum(pre, 0.0)
    s2_ref[...] = jnp.dot(h1, w2_ref[...],
                          preferred_element_type=jnp.float32) * dinv


def _mid_kernel(acc1, s1, dinv, b1, w2):
    return pl.pallas_call(
        _mid_body,
        out_shape=jax.ShapeDtypeStruct((N, D_OUT), jnp.float32),
    )(acc1, s1, dinv, b1, w2)


def _out_body(acc_ref, s2_ref, dinv_ref, b2_ref, batch_ref, hs_ref, h_ref):
    acc = acc_ref[...]
    h = (acc[0:N] + acc[NPAD:NPAD + N] + s2_ref[...]) * dinv_ref[...] \
        + b2_ref[...]
    h_ref[...] = h
    iota = lax.broadcasted_iota(jnp.int32, (G, N), 0)
    onehot_t = (batch_ref[...] == iota).astype(jnp.float32)
    hs_ref[...] = jnp.dot(onehot_t, h, preferred_element_type=jnp.float32)


def _out_kernel(acc2, s2, dinv, b2, batch_row):
    return pl.pallas_call(
        _out_body,
        out_shape=[jax.ShapeDtypeStruct((G, D_OUT), jnp.float32),
                   jax.ShapeDtypeStruct((N, D_OUT), jnp.float32)],
    )(acc2, s2, dinv, b2, batch_row)


def kernel(x, edge_index, batch, W1, b1, W2, b2):
    src3d = edge_index[0].reshape(NW, NCHUNK, CH)
    dst3d = edge_index[1].reshape(NW, NCHUNK, CH)
    src3d1 = edge_index[0].reshape(NW, NCHUNK1, CH1)
    dst3d1 = edge_index[1].reshape(NW, NCHUNK1, CH1)
    dst_flat = edge_index[1]

    deg_parts = _deg_kernel(dst_flat).reshape(NW, N)
    s1, dinv = _prep_kernel(x, W1, deg_parts)
    acc1 = _edge_kernel_h(s1, src3d1, dst3d1)
    s2 = _mid_kernel(acc1, s1, dinv, b1.reshape(1, HID), W2)
    acc2 = _edge_kernel_o(s2, src3d, dst3d)
    hs, h = _out_kernel(acc2, s2, dinv, b2.reshape(1, D_OUT),
                        batch.reshape(1, N))
    return (hs, h)


# speedup vs baseline: 35.5324x; 1.0097x over previous
import dataclasses
import functools

import jax
import jax.numpy as jnp
from jax import lax
from jax.experimental import pallas as pl
from jax.experimental.pallas import tpu as pltpu
from jax.experimental.pallas import tpu_sc as plsc

N = 10000
E = 320000
D_IN = 128
HID = 128
D_OUT = 64
G = 128

NC = 2
NS = 16
NW = NC * NS
EPW = E // NW
CH = 100
NCHUNK = EPW // CH
CH1 = 50
NCHUNK1 = EPW // CH1
NBUF = 4
NPAD = 10240
RPT = NPAD // NS

_mesh = plsc.VectorSubcoreMesh(core_axis_name="c", subcore_axis_name="s")

_sc_params = pltpu.CompilerParams()
if "needs_layout_passes" in pltpu.CompilerParams.__dataclass_fields__:
    _sc_params = dataclasses.replace(_sc_params, needs_layout_passes=False)
_sc_flat_params = _sc_params
if "use_tc_tiling_on_sc" in pltpu.CompilerParams.__dataclass_fields__:
    _sc_flat_params = dataclasses.replace(_sc_flat_params,
                                          use_tc_tiling_on_sc=False)


def _deg_body(dst_hbm, out_hbm, idx_v, deg_v):
    cid = lax.axis_index("c")
    sid = lax.axis_index("s")
    wid = cid * NS + sid
    pltpu.sync_copy(dst_hbm.at[pl.ds(wid * EPW, EPW)], idx_v)
    zeros16 = jnp.zeros((16,), jnp.float32)

    @pl.loop(0, N // 16)
    def _zero(j):
        deg_v[pl.ds(j * 16, 16)] = zeros16

    ones16 = jnp.ones((16,), jnp.float32)

    @pl.loop(0, EPW // 16)
    def _count(j):
        idx16 = idx_v[pl.ds(j * 16, 16)]
        plsc.addupdate_scatter(deg_v, [idx16], ones16)

    pltpu.sync_copy(deg_v, out_hbm.at[pl.ds(wid * N, N)])


_deg_kernel = pl.kernel(
    out_type=jax.ShapeDtypeStruct((NW * N,), jnp.float32),
    mesh=_mesh,
    compiler_params=_sc_params,
    scratch_types=[
        pltpu.VMEM((EPW,), jnp.int32),
        pltpu.VMEM((N,), jnp.float32),
    ],
)(_deg_body)


def _edge_body(d, ch, nchunk, s_hbm, src_hbm, dst_hbm, out_hbm,
               src_v, dst_v, rows, sems, acc_sh):
    cid = lax.axis_index("c")
    sid = lax.axis_index("s")
    wid = cid * NS + sid
    zeros16 = jnp.zeros((16,), jnp.float32)

    @pl.loop(0, ch)
    def _zr(r):
        @pl.loop(0, d // 16)
        def _zc(c):
            rows[0][r, pl.ds(c * 16, 16)] = zeros16

    for k in range(RPT // ch):
        pltpu.sync_copy(rows[0], acc_sh.at[pl.ds(sid * RPT + k * ch, ch)])
    rem = RPT % ch
    pltpu.sync_copy(rows[0].at[pl.ds(0, rem)],
                    acc_sh.at[pl.ds(sid * RPT + RPT - rem, rem)])
    pltpu.sync_copy(src_hbm.at[wid], src_v)
    pltpu.sync_copy(dst_hbm.at[wid], dst_v)
    plsc.subcore_barrier()

    nbuf = len(rows)
    for b in range(nbuf - 1):
        pltpu.async_copy(s_hbm.at[src_v.at[b]], rows[b], sems[b])

    @pl.loop(0, nchunk // nbuf)
    def _edges(p):
        j = p * nbuf
        for b in range(nbuf):
            pltpu.make_async_copy(s_hbm.at[src_v.at[j + b]],
                                  rows[b], sems[b]).wait()
            nxt = j + b + nbuf - 1
            bb = (b + nbuf - 1) % nbuf

            @pl.when(nxt < nchunk)
            def _issue():
                pltpu.async_copy(s_hbm.at[src_v.at[nxt]],
                                 rows[bb], sems[bb])

            pltpu.sync_copy(rows[b], acc_sh.at[dst_v.at[j + b]], add=True)

    for c in range((nchunk // nbuf) * nbuf, nchunk):
        pltpu.make_async_copy(s_hbm.at[src_v.at[c]],
                              rows[c % nbuf], sems[c % nbuf]).wait()
        pltpu.sync_copy(rows[c % nbuf], acc_sh.at[dst_v.at[c]], add=True)

    plsc.subcore_barrier()
    pltpu.sync_copy(acc_sh.at[pl.ds(sid * RPT, RPT)],
                    out_hbm.at[pl.ds(cid * NPAD + sid * RPT, RPT)])


def _make_edge_kernel(d):
    def body(s_hbm, src_hbm, dst_hbm, out_hbm, src_v, dst_v, *rest):
        rows = list(rest[:NBUF])
        acc_sh = rest[NBUF]
        sems = list(rest[NBUF + 1:NBUF + 1 + NBUF])
        _edge_body(d, CH1, NCHUNK1, s_hbm, src_hbm, dst_hbm, out_hbm,
                   src_v, dst_v, rows, sems, acc_sh)

    return pl.kernel(
        out_type=jax.ShapeDtypeStruct((NC * NPAD, d), jnp.float32),
        mesh=_mesh,
        compiler_params=_sc_flat_params,
        scratch_types=[
            pltpu.VMEM((NCHUNK1, CH1), jnp.int32),
            pltpu.VMEM((NCHUNK1, CH1), jnp.int32),
        ] + [pltpu.VMEM((CH1, d), jnp.float32) for _ in range(NBUF)] + [
            pltpu.VMEM_SHARED((NPAD, d), jnp.float32),
        ] + [pltpu.SemaphoreType.DMA for _ in range(NBUF)],
    )(body)


def _edge_body_staged(d, s_hbm, src_hbm, dst_hbm, out_hbm,
                      src_v, dst_v, *rest):
    rows = list(rest[:NBUF])
    s_sh = rest[NBUF]
    acc_sh = rest[NBUF + 1]
    sems = list(rest[NBUF + 2:NBUF + 2 + NBUF])
    cid = lax.axis_index("c")
    sid = lax.axis_index("s")
    wid = cid * NS + sid
    zeros16 = jnp.zeros((16,), jnp.float32)

    @pl.loop(0, CH)
    def _zr(r):
        @pl.loop(0, d // 16)
        def _zc(c):
            rows[0][r, pl.ds(c * 16, 16)] = zeros16

    for k in range(RPT // CH):
        pltpu.sync_copy(rows[0], acc_sh.at[pl.ds(sid * RPT + k * CH, CH)])
    rem = RPT % CH
    pltpu.sync_copy(rows[0].at[pl.ds(0, rem)],
                    acc_sh.at[pl.ds(sid * RPT + RPT - rem, rem)])
    pltpu.sync_copy(s_hbm.at[pl.ds(sid * (N // NS), N // NS)],
                    s_sh.at[pl.ds(sid * (N // NS), N // NS)])
    pltpu.sync_copy(src_hbm.at[wid], src_v)
    pltpu.sync_copy(dst_hbm.at[wid], dst_v)
    plsc.subcore_barrier()

    nbuf = len(rows)
    for b in range(nbuf - 1):
        pltpu.async_copy(s_sh.at[src_v.at[b]], rows[b], sems[b])

    @pl.loop(0, NCHUNK // nbuf)
    def _edges(p):
        j = p * nbuf
        for b in range(nbuf):
            pltpu.make_async_copy(s_sh.at[src_v.at[j + b]],
                                  rows[b], sems[b]).wait()
            nxt = j + b + nbuf - 1
            bb = (b + nbuf - 1) % nbuf

            @pl.when(nxt < NCHUNK)
            def _issue():
                pltpu.async_copy(s_sh.at[src_v.at[nxt]],
                                 rows[bb], sems[bb])

            pltpu.sync_copy(rows[b], acc_sh.at[dst_v.at[j + b]], add=True)

    for c in range((NCHUNK // nbuf) * nbuf, NCHUNK):
        pltpu.make_async_copy(s_sh.at[src_v.at[c]],
                              rows[c % nbuf], sems[c % nbuf]).wait()
        pltpu.sync_copy(rows[c % nbuf], acc_sh.at[dst_v.at[c]], add=True)

    plsc.subcore_barrier()
    pltpu.sync_copy(acc_sh.at[pl.ds(sid * RPT, RPT)],
                    out_hbm.at[pl.ds(cid * NPAD + sid * RPT, RPT)])


def _make_edge_kernel_staged(d):
    return pl.kernel(
        out_type=jax.ShapeDtypeStruct((NC * NPAD, d), jnp.float32),
        mesh=_mesh,
        compiler_params=_sc_flat_params,
        scratch_types=[
            pltpu.VMEM((NCHUNK, CH), jnp.int32),
            pltpu.VMEM((NCHUNK, CH), jnp.int32),
        ] + [pltpu.VMEM((CH, d), jnp.float32) for _ in range(NBUF)] + [
            pltpu.VMEM_SHARED((N, d), jnp.float32),
            pltpu.VMEM_SHARED((NPAD, d), jnp.float32),
        ] + [pltpu.SemaphoreType.DMA for _ in range(NBUF)],
    )(functools.partial(_edge_body_staged, d))


_edge_kernel_h = _make_edge_kernel(HID)
_edge_kernel_o = _make_edge_kernel_staged(D_OUT)


def _prep_body(x_ref, w_ref, parts_ref, s_ref, dinv_ref):
    xw = jnp.dot(x_ref[...], w_ref[...], preferred_element_type=jnp.float32)
    ones_col = jnp.ones((NW, 1), jnp.float32)
    deg = lax.dot_general(parts_ref[...], ones_col, (((0,), (0,)), ((), ())),
                          preferred_element_type=jnp.float32) + 1.0
    dinv = lax.rsqrt(deg)
    dinv_ref[...] = dinv
    s_ref[...] = xw * dinv


def _prep_kernel(x, w1, parts):
    return pl.pallas_call(
        _prep_body,
        out_shape=[jax.ShapeDtypeStruct((N, HID), jnp.float32),
                   jax.ShapeDtypeStruct((N, 1), jnp.float32)],
    )(x, w1, parts)


def _mid_body(acc_ref, s1_ref, dinv_ref, b1_ref, w2_ref, s2_ref):
    dinv = dinv_ref[...]
    acc = acc_ref[...]
    pre = (acc[0:N] + acc[NPAD:NPAD + N] + s1_ref[...]) * dinv + b1_ref[...]
    h1 = jnp.maximum(pre, 0.0)
    s2_ref[...] = jnp.dot(h1, w2_ref[...],
                          preferred_element_type=jnp.float32) * dinv


def _mid_kernel(acc1, s1, dinv, b1, w2):
    return pl.pallas_call(
        _mid_body,
        out_shape=jax.ShapeDtypeStruct((N, D_OUT), jnp.float32),
    )(acc1, s1, dinv, b1, w2)


def _out_body(acc_ref, s2_ref, dinv_ref, b2_ref, batch_ref, hs_ref, h_ref):
    acc = acc_ref[...]
    h = (acc[0:N] + acc[NPAD:NPAD + N] + s2_ref[...]) * dinv_ref[...] \
        + b2_ref[...]
    h_ref[...] = h
    iota = lax.broadcasted_iota(jnp.int32, (G, N), 0)
    onehot_t = (batch_ref[...] == iota).astype(jnp.float32)
    hs_ref[...] = jnp.dot(onehot_t, h, preferred_element_type=jnp.float32)


def _out_kernel(acc2, s2, dinv, b2, batch_row):
    return pl.pallas_call(
        _out_body,
        out_shape=[jax.ShapeDtypeStruct((G, D_OUT), jnp.float32),
                   jax.ShapeDtypeStruct((N, D_OUT), jnp.float32)],
    )(acc2, s2, dinv, b2, batch_row)


def kernel(x, edge_index, batch, W1, b1, W2, b2):
    src3d = edge_index[0].reshape(NW, NCHUNK, CH)
    dst3d = edge_index[1].reshape(NW, NCHUNK, CH)
    src3d1 = edge_index[0].reshape(NW, NCHUNK1, CH1)
    dst3d1 = edge_index[1].reshape(NW, NCHUNK1, CH1)
    dst_flat = edge_index[1]

    deg_parts = _deg_kernel(dst_flat).reshape(NW, N)
    s1, dinv = _prep_kernel(x, W1, deg_parts)
    acc1 = _edge_kernel_h(s1, src3d1, dst3d1)
    s2 = _mid_kernel(acc1, s1, dinv, b1.reshape(1, HID), W2)
    acc2 = _edge_kernel_o(s2, src3d, dst3d)
    hs, h = _out_kernel(acc2, s2, dinv, b2.reshape(1, D_OUT),
                        batch.reshape(1, N))
    return (hs, h)
